# baseline (speedup 1.0000x reference)
import functools

import jax
import jax.numpy as jnp
from jax import lax
from jax.experimental import pallas as pl
from jax.experimental.pallas import tpu as pltpu
from jax.experimental.pallas import tpu_sc as plsc

B = 16384
NF = 16
NN = 63
NNP = 64
EMB = 128
NR = NF + 1

_info = plsc.get_sparse_core_info()
NC = _info.num_cores
NS = _info.num_subcores
NW = NC * NS
SPW = B // NW
CH = 16
CB = CH // 8
NCHUNK = SPW // CH

TB = 4096

_mesh = plsc.VectorSubcoreMesh(core_axis_name="c", subcore_axis_name="s")


def _lin_body(x_ref, w_ref, b_ref, cat_ref, o_ref, catp_ref):
    o_ref[...] = (
        lax.dot_general(x_ref[...], w_ref[...], (((1,), (1,)), ((), ())),
                        preferred_element_type=jnp.float32)
        + b_ref[...]
    )
    c = cat_ref[...].reshape(TB // 8, 8, NF)
    catp_ref[...] = c.transpose(0, 2, 1).reshape(TB // 8, NF * 8)


_lin_kernel = pl.pallas_call(
    _lin_body,
    out_shape=(
        jax.ShapeDtypeStruct((B, EMB), jnp.float32),
        jax.ShapeDtypeStruct((B // 8, NF * 8), jnp.int32),
    ),
    grid=(B // TB,),
    in_specs=[
        pl.BlockSpec((TB, NN), lambda i: (i, 0)),
        pl.BlockSpec((EMB, NN), lambda i: (0, 0)),
        pl.BlockSpec((EMB,), lambda i: (0,)),
        pl.BlockSpec((TB, NF), lambda i: (i, 0)),
    ],
    out_specs=(
        pl.BlockSpec((TB, EMB), lambda i: (i, 0)),
        pl.BlockSpec((TB // 8, NF * 8), lambda i: (i, 0)),
    ),
)


@functools.partial(
    pl.kernel,
    out_type=jax.ShapeDtypeStruct((B // 8, NR * 8, EMB), jnp.float32),
    mesh=_mesh,
    scratch_types=[
        pltpu.VMEM((SPW // 8, NF * 8), jnp.int32),
        pltpu.VMEM((4, NR * 8, EMB), jnp.float32),
        pltpu.VMEM_SHARED((128, EMB), jnp.float32),
        pltpu.SemaphoreType.DMA,
        pltpu.SemaphoreType.DMA,
        pltpu.SemaphoreType.DMA,
    ],
)
def _emb_kernel(catp_hbm, nemb_hbm, tbl_hbm, out_hbm,
                idx_v, obuf, tbl_sh, g_sem, n_sem, out_sem):
    wid = lax.axis_index("c") * NS + lax.axis_index("s")
    base0 = pl.multiple_of(wid * SPW, 8)
    blk0 = pl.multiple_of(wid * (SPW // 8), 8)

    @pl.when(lax.axis_index("s") == 0)
    def _():
        pltpu.sync_copy(tbl_hbm, tbl_sh.at[pl.ds(0, 127)])

    pltpu.sync_copy(catp_hbm.at[pl.ds(blk0, SPW // 8)], idx_v)
    plsc.subcore_barrier()

    NBLK = SPW // 8

    def gather_copy(blk, p):
        return pltpu.make_async_copy(
            tbl_sh.at[idx_v.at[blk]],
            obuf.at[p, pl.ds(0, NF * 8)], g_sem)

    def nemb_copy(blk, p):
        return pltpu.make_async_copy(
            nemb_hbm.at[pl.ds(pl.multiple_of(base0 + blk * 8, 8), 8)],
            obuf.at[p, pl.ds(NF * 8, 8)], n_sem)

    def out_copy(blk, p):
        return pltpu.make_async_copy(
            obuf.at[p], out_hbm.at[blk0 + blk], out_sem)

    def blk_body(blk, carry):
        p = lax.rem(blk, 4)

        @pl.when(blk >= 4)
        def _():
            out_copy(blk - 4, p).wait()

        gather_copy(blk, p).start()
        nemb_copy(blk, p).start()

        @pl.when(blk >= 1)
        def _():
            pm = lax.rem(blk + 3, 4)
            gather_copy(blk - 1, pm).wait()
            nemb_copy(blk - 1, pm).wait()
            out_copy(blk - 1, pm).start()
        return carry

    lax.fori_loop(0, NBLK, blk_body, 0)

    pl_last = lax.rem(NBLK - 1, 4)
    gather_copy(NBLK - 1, pl_last).wait()
    nemb_copy(NBLK - 1, pl_last).wait()
    out_copy(NBLK - 1, pl_last).start()
    for k in range(4):
        out_copy(NBLK - 4 + k, lax.rem(NBLK - 4 + k, 4)).wait()


def kernel(categorical_x, numerical_x, emb_table, W, b):
    cat = categorical_x.astype(jnp.int32)
    nemb, catp = _lin_kernel(numerical_x, W, b, cat)
    out4 = _emb_kernel(catp, nemb, emb_table)
    out4 = out4.reshape(B // 8, NR, 8, EMB)
    return out4.transpose(0, 2, 1, 3).reshape(B, NR * EMB)


# speedup vs baseline: 1.0010x; 1.0010x over previous
import functools

import jax
import jax.numpy as jnp
from jax import lax
from jax.experimental import pallas as pl
from jax.experimental.pallas import tpu as pltpu
from jax.experimental.pallas import tpu_sc as plsc

B = 16384
NF = 16
NN = 63
NNP = 64
EMB = 128
NR = NF + 1

_info = plsc.get_sparse_core_info()
NC = _info.num_cores
NS = _info.num_subcores
NW = NC * NS
SPW = B // NW
CH = 16
CB = CH // 8
NCHUNK = SPW // CH

TB = 4096

_mesh = plsc.VectorSubcoreMesh(core_axis_name="c", subcore_axis_name="s")


def _lin_body(x_ref, w_ref, b_ref, cat_ref, o_ref, catp_ref):
    o_ref[...] = (
        lax.dot_general(x_ref[...], w_ref[...], (((1,), (1,)), ((), ())),
                        preferred_element_type=jnp.float32)
        + b_ref[...]
    )
    c = cat_ref[...].reshape(TB // 8, 8, NF)
    catp_ref[...] = c.transpose(0, 2, 1).reshape(TB // 8, NF * 8)


_lin_kernel = pl.pallas_call(
    _lin_body,
    out_shape=(
        jax.ShapeDtypeStruct((B, EMB), jnp.float32),
        jax.ShapeDtypeStruct((B // 8, NF * 8), jnp.int32),
    ),
    grid=(B // TB,),
    in_specs=[
        pl.BlockSpec((TB, NN), lambda i: (i, 0)),
        pl.BlockSpec((EMB, NN), lambda i: (0, 0)),
        pl.BlockSpec((EMB,), lambda i: (0,)),
        pl.BlockSpec((TB, NF), lambda i: (i, 0)),
    ],
    out_specs=(
        pl.BlockSpec((TB, EMB), lambda i: (i, 0)),
        pl.BlockSpec((TB // 8, NF * 8), lambda i: (i, 0)),
    ),
)


@functools.partial(
    pl.kernel,
    out_type=jax.ShapeDtypeStruct((B // 8, NR * 8, EMB), jnp.float32),
    mesh=_mesh,
    scratch_types=[
        pltpu.VMEM((SPW // 8, NF * 8), jnp.int32),
        pltpu.VMEM((6, NR * 8, EMB), jnp.float32),
        pltpu.VMEM_SHARED((128, EMB), jnp.float32),
        pltpu.SemaphoreType.DMA,
        pltpu.SemaphoreType.DMA,
        pltpu.SemaphoreType.DMA,
    ],
)
def _emb_kernel(catp_hbm, nemb_hbm, tbl_hbm, out_hbm,
                idx_v, obuf, tbl_sh, g_sem, n_sem, out_sem):
    wid = lax.axis_index("c") * NS + lax.axis_index("s")
    base0 = pl.multiple_of(wid * SPW, 8)
    blk0 = pl.multiple_of(wid * (SPW // 8), 8)

    @pl.when(lax.axis_index("s") == 0)
    def _():
        pltpu.sync_copy(tbl_hbm, tbl_sh.at[pl.ds(0, 127)])

    pltpu.sync_copy(catp_hbm.at[pl.ds(blk0, SPW // 8)], idx_v)
    plsc.subcore_barrier()

    NBLK = SPW // 8

    def gather_copy(blk, p):
        return pltpu.make_async_copy(
            tbl_sh.at[idx_v.at[blk]],
            obuf.at[p, pl.ds(0, NF * 8)], g_sem)

    def nemb_copy(blk, p):
        return pltpu.make_async_copy(
            nemb_hbm.at[pl.ds(pl.multiple_of(base0 + blk * 8, 8), 8)],
            obuf.at[p, pl.ds(NF * 8, 8)], n_sem)

    def out_copy(blk, p):
        return pltpu.make_async_copy(
            obuf.at[p], out_hbm.at[blk0 + blk], out_sem)

    def blk_body(blk, carry):
        p = lax.rem(blk, 6)

        @pl.when(blk >= 6)
        def _():
            out_copy(blk - 6, p).wait()

        gather_copy(blk, p).start()
        nemb_copy(blk, p).start()

        @pl.when(blk >= 1)
        def _():
            pm = lax.rem(blk + 5, 6)
            gather_copy(blk - 1, pm).wait()
            nemb_copy(blk - 1, pm).wait()
            out_copy(blk - 1, pm).start()
        return carry

    lax.fori_loop(0, NBLK, blk_body, 0)

    pl_last = lax.rem(NBLK - 1, 6)
    gather_copy(NBLK - 1, pl_last).wait()
    nemb_copy(NBLK - 1, pl_last).wait()
    out_copy(NBLK - 1, pl_last).start()
    for k in range(6):
        out_copy(NBLK - 6 + k, lax.rem(NBLK - 6 + k, 6)).wait()


def kernel(categorical_x, numerical_x, emb_table, W, b):
    cat = categorical_x.astype(jnp.int32)
    nemb, catp = _lin_kernel(numerical_x, W, b, cat)
    out4 = _emb_kernel(catp, nemb, emb_table)
    out4 = out4.reshape(B // 8, NR, 8, EMB)
    return out4.transpose(0, 2, 1, 3).reshape(B, NR * EMB)


# speedup vs baseline: 1.0020x; 1.0010x over previous
import functools

import jax
import jax.numpy as jnp
from jax import lax
from jax.experimental import pallas as pl
from jax.experimental.pallas import tpu as pltpu
from jax.experimental.pallas import tpu_sc as plsc

B = 16384
NF = 16
NN = 63
EMB = 128
NR = NF + 1

_info = plsc.get_sparse_core_info()
NC = _info.num_cores
NS = _info.num_subcores
NW = NC * NS
SPW = B // NW
NRING = 6

TB = 4096

_mesh = plsc.VectorSubcoreMesh(core_axis_name="c", subcore_axis_name="s")


def _lin_body(x_ref, w_ref, b_ref, cat_ref, o_ref, catp_ref):
    o_ref[...] = (
        lax.dot_general(x_ref[...], w_ref[...], (((1,), (1,)), ((), ())),
                        preferred_element_type=jnp.float32)
        + b_ref[...]
    )
    c = cat_ref[...].reshape(TB // 8, 8, NF)
    catp_ref[...] = c.transpose(0, 2, 1).reshape(TB // 8, NF * 8)


_lin_kernel = pl.pallas_call(
    _lin_body,
    out_shape=(
        jax.ShapeDtypeStruct((B, EMB), jnp.float32),
        jax.ShapeDtypeStruct((B // 8, NF * 8), jnp.int32),
    ),
    grid=(B // TB,),
    in_specs=[
        pl.BlockSpec((TB, NN), lambda i: (i, 0)),
        pl.BlockSpec((EMB, NN), lambda i: (0, 0)),
        pl.BlockSpec((EMB,), lambda i: (0,)),
        pl.BlockSpec((TB, NF), lambda i: (i, 0)),
    ],
    out_specs=(
        pl.BlockSpec((TB, EMB), lambda i: (i, 0)),
        pl.BlockSpec((TB // 8, NF * 8), lambda i: (i, 0)),
    ),
)


@functools.partial(
    pl.kernel,
    out_type=jax.ShapeDtypeStruct((B // 8, NR * 8, EMB), jnp.float32),
    mesh=_mesh,
    scratch_types=[
        pltpu.VMEM((SPW // 8, NF * 8), jnp.int32),
        pltpu.VMEM((NRING, NR * 8, EMB), jnp.float32),
        pltpu.VMEM_SHARED((128, EMB), jnp.float32),
        pltpu.SemaphoreType.DMA,
        pltpu.SemaphoreType.DMA,
        pltpu.SemaphoreType.DMA,
    ],
)
def _emb_kernel(catp_hbm, nemb_hbm, tbl_hbm, out_hbm,
                idx_v, obuf, tbl_sh, g_sem, n_sem, out_sem):
    wid = lax.axis_index("c") * NS + lax.axis_index("s")
    base0 = pl.multiple_of(wid * SPW, 8)
    blk0 = pl.multiple_of(wid * (SPW // 8), 8)

    @pl.when(lax.axis_index("s") == 0)
    def _():
        pltpu.sync_copy(tbl_hbm, tbl_sh.at[pl.ds(0, 127)])

    pltpu.sync_copy(catp_hbm.at[pl.ds(blk0, SPW // 8)], idx_v)
    plsc.subcore_barrier()

    NBLK = SPW // 8

    def gather_copy(blk, p):
        return pltpu.make_async_copy(
            tbl_sh.at[idx_v.at[blk]],
            obuf.at[p, pl.ds(0, NF * 8)], g_sem)

    def nemb_copy(blk, p):
        return pltpu.make_async_copy(
            nemb_hbm.at[pl.ds(pl.multiple_of(base0 + blk * 8, 8), 8)],
            obuf.at[p, pl.ds(NF * 8, 8)], n_sem)

    def out_copy(blk, p):
        return pltpu.make_async_copy(
            obuf.at[p], out_hbm.at[blk0 + blk], out_sem)

    def blk_body(blk, carry):
        p = lax.rem(blk, NRING)

        @pl.when(blk >= NRING)
        def _():
            out_copy(blk - NRING, p).wait()

        gather_copy(blk, p).start()
        nemb_copy(blk, p).start()

        @pl.when(blk >= 1)
        def _():
            pm = lax.rem(blk + NRING - 1, NRING)
            gather_copy(blk - 1, pm).wait()
            nemb_copy(blk - 1, pm).wait()
            out_copy(blk - 1, pm).start()
        return carry

    lax.fori_loop(0, NBLK, blk_body, 0)

    pl_last = lax.rem(NBLK - 1, NRING)
    gather_copy(NBLK - 1, pl_last).wait()
    nemb_copy(NBLK - 1, pl_last).wait()
    out_copy(NBLK - 1, pl_last).start()
    for k in range(NRING):
        out_copy(NBLK - NRING + k, lax.rem(NBLK - NRING + k, NRING)).wait()


def kernel(categorical_x, numerical_x, emb_table, W, b):
    cat = categorical_x.astype(jnp.int32)
    nemb, catp = _lin_kernel(numerical_x, W, b, cat)
    out4 = _emb_kernel(catp, nemb, emb_table)
    out4 = out4.reshape(B // 8, NR, 8, EMB)
    return out4.transpose(0, 2, 1, 3).reshape(B, NR * EMB)
